# Initial kernel scaffold; baseline (speedup 1.0000x reference)
import functools

import jax
import jax.numpy as jnp
from jax import lax
from jax.experimental import pallas as pl
from jax.experimental.pallas import tpu as pltpu
from jax.experimental.pallas import tpu_sc as plsc

NC = 2
NS = 16
NW = NC * NS
CHUNK = 128

_MESH = plsc.VectorSubcoreMesh(core_axis_name="c", subcore_axis_name="s")


def _make_prop(np_rows, d, ep_w, n_chunks):
    rpt = np_rows // NS

    @functools.partial(
        pl.kernel,
        out_type=jax.ShapeDtypeStruct((2, np_rows, d), jnp.float32),
        mesh=_MESH,
        scratch_types=[
            pltpu.VMEM((CHUNK,), jnp.int32),
            pltpu.VMEM((CHUNK,), jnp.int32),
            pltpu.VMEM((CHUNK, d), jnp.float32),
            pltpu.VMEM_SHARED((np_rows, d), jnp.float32),
            pltpu.SemaphoreType.DMA,
        ],
    )
    def prop(xin, srcp, dstp, out, src_v, dst_v, rows_v, acc, sem):
        cid = lax.axis_index("c")
        sid = lax.axis_index("s")
        r0 = sid * rpt

        def init_body(i, carry):
            rb = r0 + i * CHUNK
            pltpu.sync_copy(xin.at[pl.ds(rb, CHUNK)], rows_v)
            pltpu.sync_copy(rows_v, acc.at[pl.ds(rb, CHUNK)])
            return carry

        lax.fori_loop(0, rpt // CHUNK, init_body, 0)
        plsc.subcore_barrier()

        base = (cid * NS + sid) * ep_w

        def edge_body(i, carry):
            eb = base + i * CHUNK
            pltpu.sync_copy(srcp.at[pl.ds(eb, CHUNK)], src_v)
            pltpu.sync_copy(dstp.at[pl.ds(eb, CHUNK)], dst_v)
            pltpu.async_copy(xin.at[src_v], rows_v, sem).wait()
            pltpu.sync_copy(rows_v, acc.at[dst_v], add=True)
            return carry

        lax.fori_loop(0, n_chunks, edge_body, 0)
        plsc.subcore_barrier()

        def out_body(i, carry):
            rb = r0 + i * CHUNK
            pltpu.sync_copy(acc.at[pl.ds(rb, CHUNK)], rows_v)
            pltpu.sync_copy(rows_v, out.at[cid, pl.ds(rb, CHUNK)])
            return carry

        lax.fori_loop(0, rpt // CHUNK, out_body, 0)

    return prop


def _make_deg(np_rows, degw, ep_w, n_chunks):
    rpt = np_rows // NS

    @functools.partial(
        pl.kernel,
        out_type=jax.ShapeDtypeStruct((2, np_rows, degw), jnp.float32),
        mesh=_MESH,
        scratch_types=[
            pltpu.VMEM((CHUNK,), jnp.int32),
            pltpu.VMEM((CHUNK, degw), jnp.float32),
            pltpu.VMEM_SHARED((np_rows, degw), jnp.float32),
            pltpu.SemaphoreType.DMA,
        ],
    )
    def deg(ones_hbm, dstp, out, dst_v, ones_v, acc, sem):
        cid = lax.axis_index("c")
        sid = lax.axis_index("s")
        r0 = sid * rpt
        pltpu.sync_copy(ones_hbm, ones_v)

        def init_body(i, carry):
            pltpu.sync_copy(ones_v, acc.at[pl.ds(r0 + i * CHUNK, CHUNK)])
            return carry

        lax.fori_loop(0, rpt // CHUNK, init_body, 0)
        plsc.subcore_barrier()

        base = (cid * NS + sid) * ep_w

        def edge_body(i, carry):
            pltpu.sync_copy(dstp.at[pl.ds(base + i * CHUNK, CHUNK)], dst_v)
            pltpu.sync_copy(ones_v, acc.at[dst_v], add=True)
            return carry

        lax.fori_loop(0, n_chunks, edge_body, 0)
        plsc.subcore_barrier()

        def out_body(i, carry):
            rb = r0 + i * CHUNK
            pltpu.sync_copy(acc.at[pl.ds(rb, CHUNK)], ones_v)
            pltpu.sync_copy(ones_v, out.at[cid, pl.ds(rb, CHUNK)])
            return carry

        lax.fori_loop(0, rpt // CHUNK, out_body, 0)

    return deg


def _pad2(a, rows, cols):
    return jnp.pad(a, ((0, rows - a.shape[0]), (0, cols - a.shape[1])))


def kernel(x, edge_index, eps, W1, b1, W2, b2, W3, b3, W4, b4,
           W_mu, b_mu, W_ls, b_ls):
    n = x.shape[0]
    e = edge_index.shape[1]
    f_in = x.shape[1]
    d2p = (W2.shape[1] + 15) // 16 * 16
    d3p = (W3.shape[1] + 15) // 16 * 16
    d4p = (W4.shape[1] + 15) // 16 * 16
    k_out = W_mu.shape[1]
    np_rows = (n + 16 * CHUNK - 1) // (16 * CHUNK) * (16 * CHUNK)
    br = 1024
    grid = (np_rows // br,)

    ep_w = ((e + NW - 1) // NW + CHUNK - 1) // CHUNK * CHUNK
    e_pad = ep_w * NW
    pad = e_pad - e
    pad_i = jnp.arange(pad, dtype=jnp.int32)
    srcp = jnp.concatenate([edge_index[0], n + (pad_i % 8)])
    dstp = jnp.concatenate([edge_index[1], n + 8 + (pad_i % 32)])
    xpad = _pad2(x, np_rows, f_in)
    eps_p = _pad2(eps, np_rows, k_out)
    W2p = _pad2(W2, W2.shape[0], d2p)
    W3p = _pad2(W3, d2p, d3p)
    W4p = _pad2(W4, d3p, d4p)
    W_mup = _pad2(W_mu, d4p, k_out)
    W_lsp = _pad2(W_ls, d4p, k_out)
    b1r = b1[None, :]
    b2p = _pad2(b2[None, :], 1, d2p)
    b3p = _pad2(b3[None, :], 1, d3p)
    b4p = _pad2(b4[None, :], 1, d4p)
    ones = jnp.ones((CHUNK, 8), dtype=jnp.float32)
    n_chunks = ep_w // CHUNK

    row_spec = lambda w: pl.BlockSpec((br, w), lambda i: (i, 0))
    parts_spec = lambda w: pl.BlockSpec((2, br, w), lambda i: (0, i, 0))
    full_spec = lambda a: pl.BlockSpec(a.shape, lambda i: (0,) * a.ndim)
    out_row = lambda w: jax.ShapeDtypeStruct((np_rows, w), jnp.float32)

    degp = _make_deg(np_rows, 8, ep_w, n_chunks)(ones, dstp)

    def tc0(degp_r, x_r, dinv_r, xs_r):
        i = pl.program_id(0)
        deg = degp_r[0, :, 0:1] + degp_r[1, :, 0:1] - 1.0
        rows = i * br + lax.broadcasted_iota(jnp.int32, (br, 1), 0)
        dinv = jnp.where(rows < n, lax.rsqrt(deg), 0.0)
        dinv_r[...] = dinv
        xs_r[...] = x_r[...] * dinv

    dinv, xs1 = pl.pallas_call(
        tc0, grid=grid,
        in_specs=[parts_spec(8), row_spec(f_in)],
        out_specs=[row_spec(1), row_spec(f_in)],
        out_shape=[out_row(1), out_row(f_in)],
    )(degp, xpad)

    p1 = _make_prop(np_rows, f_in, ep_w, n_chunks)(xs1, srcp, dstp)

    def tc1(dinv_r, p_r, xs_r, W1_r, b1_r, W2_r, v2_r):
        dinv = dinv_r[...]
        u = (p_r[0] + p_r[1] - xs_r[...]) * dinv
        h = jnp.maximum(
            jnp.dot(u, W1_r[...], preferred_element_type=jnp.float32)
            + b1_r[...], 0.0)
        v2_r[...] = jnp.dot(
            h, W2_r[...], preferred_element_type=jnp.float32) * dinv

    v2 = pl.pallas_call(
        tc1, grid=grid,
        in_specs=[row_spec(1), parts_spec(f_in), row_spec(f_in),
                  full_spec(W1), full_spec(b1r), full_spec(W2p)],
        out_specs=row_spec(d2p),
        out_shape=out_row(d2p),
    )(dinv, p1, xs1, W1, b1r, W2p)

    def mid(v, b_r, W_r, w_in, w_out):
        p = _make_prop(np_rows, w_in, ep_w, n_chunks)(v, srcp, dstp)

        def tc(dinv_r, p_r, v_r, b_rr, W_rr, vo_r):
            dinv = dinv_r[...]
            h = jnp.maximum((p_r[0] + p_r[1] - v_r[...]) * dinv + b_rr[...],
                            0.0)
            vo_r[...] = jnp.dot(
                h, W_rr[...], preferred_element_type=jnp.float32) * dinv

        return pl.pallas_call(
            tc, grid=grid,
            in_specs=[row_spec(1), parts_spec(w_in), row_spec(w_in),
                      full_spec(b_r), full_spec(W_r)],
            out_specs=row_spec(w_out),
            out_shape=out_row(w_out),
        )(dinv, p, v, b_r, W_r)

    v3 = mid(v2, b2p, W3p, d2p, d3p)
    v4 = mid(v3, b3p, W4p, d3p, d4p)

    p4 = _make_prop(np_rows, d4p, ep_w, n_chunks)(v4, srcp, dstp)

    def tc4(dinv_r, p_r, v_r, b_rr, v5_r):
        dinv = dinv_r[...]
        h = jnp.maximum((p_r[0] + p_r[1] - v_r[...]) * dinv + b_rr[...], 0.0)
        v5_r[...] = h * dinv

    v5 = pl.pallas_call(
        tc4, grid=grid,
        in_specs=[row_spec(1), parts_spec(d4p), row_spec(d4p),
                  full_spec(b4p)],
        out_specs=row_spec(d4p),
        out_shape=out_row(d4p),
    )(dinv, p4, v4, b4p)

    p5 = _make_prop(np_rows, d4p, ep_w, n_chunks)(v5, srcp, dstp)

    def tc5(dinv_r, p_r, v_r, Wm_r, bm_r, Wl_r, bl_r, eps_r, pz_r, z_r):
        g = (p_r[0] + p_r[1] - v_r[...]) * dinv_r[...]
        mu = jnp.dot(g, Wm_r[...], preferred_element_type=jnp.float32) \
            + bm_r[...]
        ls = jnp.dot(g, Wl_r[...], preferred_element_type=jnp.float32) \
            + bl_r[...]
        z = mu + eps_r[...] * jnp.exp(ls)
        m = jnp.max(z, axis=1, keepdims=True)
        lse = m + jnp.log(jnp.sum(jnp.exp(z - m), axis=1, keepdims=True))
        pz_r[...] = z - lse
        z_r[...] = z

    pz, z = pl.pallas_call(
        tc5, grid=grid,
        in_specs=[row_spec(1), parts_spec(d4p), row_spec(d4p),
                  full_spec(W_mup), full_spec(b_mu[None, :]),
                  full_spec(W_lsp), full_spec(b_ls[None, :]),
                  row_spec(k_out)],
        out_specs=[row_spec(k_out), row_spec(k_out)],
        out_shape=[out_row(k_out), out_row(k_out)],
    )(dinv, p5, v5, W_mup, b_mu[None, :], W_lsp, b_ls[None, :], eps_p)

    return (pz[:n], z[:n])


# speedup vs baseline: 13.8983x; 13.8983x over previous
import functools

import jax
import jax.numpy as jnp
from jax import lax
from jax.experimental import pallas as pl
from jax.experimental.pallas import tpu as pltpu
from jax.experimental.pallas import tpu_sc as plsc

NC = 2
NS = 16
NW = NC * NS
CHUNK = 128

_MESH = plsc.VectorSubcoreMesh(core_axis_name="c", subcore_axis_name="s")
_SC_PARAMS = pltpu.CompilerParams(use_tc_tiling_on_sc=False)


def _make_prop(np_rows, d, ep_w, n_chunks):
    rpt = np_rows // NS

    @functools.partial(
        pl.kernel,
        out_type=jax.ShapeDtypeStruct((2, np_rows, d), jnp.float32),
        mesh=_MESH,
        scratch_types=[
            pltpu.VMEM((CHUNK,), jnp.int32),
            pltpu.VMEM((CHUNK,), jnp.int32),
            pltpu.VMEM((CHUNK, d), jnp.float32),
            pltpu.VMEM_SHARED((np_rows, d), jnp.float32),
            pltpu.SemaphoreType.DMA,
        ],
        compiler_params=_SC_PARAMS,
    )
    def prop(xin, srcp, dstp, out, src_v, dst_v, rows_v, acc, sem):
        cid = lax.axis_index("c")
        sid = lax.axis_index("s")
        r0 = sid * rpt

        def init_body(i, carry):
            rb = r0 + i * CHUNK
            pltpu.sync_copy(xin.at[pl.ds(rb, CHUNK)], rows_v)
            pltpu.sync_copy(rows_v, acc.at[pl.ds(rb, CHUNK)])
            return carry

        lax.fori_loop(0, rpt // CHUNK, init_body, 0)
        plsc.subcore_barrier()

        base = (cid * NS + sid) * ep_w

        def edge_body(i, carry):
            eb = base + i * CHUNK
            pltpu.sync_copy(srcp.at[pl.ds(eb, CHUNK)], src_v)
            pltpu.sync_copy(dstp.at[pl.ds(eb, CHUNK)], dst_v)
            pltpu.async_copy(xin.at[src_v], rows_v, sem).wait()
            pltpu.sync_copy(rows_v, acc.at[dst_v], add=True)
            return carry

        lax.fori_loop(0, n_chunks, edge_body, 0)
        plsc.subcore_barrier()

        def out_body(i, carry):
            rb = r0 + i * CHUNK
            pltpu.sync_copy(acc.at[pl.ds(rb, CHUNK)], rows_v)
            pltpu.sync_copy(rows_v, out.at[cid, pl.ds(rb, CHUNK)])
            return carry

        lax.fori_loop(0, rpt // CHUNK, out_body, 0)

    return prop


def _make_deg(np_rows, degw, ep_w, n_chunks):
    rpt = np_rows // NS

    @functools.partial(
        pl.kernel,
        out_type=jax.ShapeDtypeStruct((2, np_rows, degw), jnp.float32),
        mesh=_MESH,
        scratch_types=[
            pltpu.VMEM((CHUNK,), jnp.int32),
            pltpu.VMEM((CHUNK, degw), jnp.float32),
            pltpu.VMEM_SHARED((np_rows, degw), jnp.float32),
            pltpu.SemaphoreType.DMA,
        ],
        compiler_params=_SC_PARAMS,
    )
    def deg(ones_hbm, dstp, out, dst_v, ones_v, acc, sem):
        cid = lax.axis_index("c")
        sid = lax.axis_index("s")
        r0 = sid * rpt
        pltpu.sync_copy(ones_hbm, ones_v)

        def init_body(i, carry):
            pltpu.sync_copy(ones_v, acc.at[pl.ds(r0 + i * CHUNK, CHUNK)])
            return carry

        lax.fori_loop(0, rpt // CHUNK, init_body, 0)
        plsc.subcore_barrier()

        base = (cid * NS + sid) * ep_w

        def edge_body(i, carry):
            pltpu.sync_copy(dstp.at[pl.ds(base + i * CHUNK, CHUNK)], dst_v)
            pltpu.sync_copy(ones_v, acc.at[dst_v], add=True)
            return carry

        lax.fori_loop(0, n_chunks, edge_body, 0)
        plsc.subcore_barrier()

        def out_body(i, carry):
            rb = r0 + i * CHUNK
            pltpu.sync_copy(acc.at[pl.ds(rb, CHUNK)], ones_v)
            pltpu.sync_copy(ones_v, out.at[cid, pl.ds(rb, CHUNK)])
            return carry

        lax.fori_loop(0, rpt // CHUNK, out_body, 0)

    return deg


def _pad2(a, rows, cols):
    return jnp.pad(a, ((0, rows - a.shape[0]), (0, cols - a.shape[1])))


def kernel(x, edge_index, eps, W1, b1, W2, b2, W3, b3, W4, b4,
           W_mu, b_mu, W_ls, b_ls):
    n = x.shape[0]
    e = edge_index.shape[1]
    f_in = x.shape[1]
    d2a = 128
    d2b = 48
    d2p = d2a + d2b
    d3p = (W3.shape[1] + 15) // 16 * 16
    d4p = (W4.shape[1] + 15) // 16 * 16
    k_out = W_mu.shape[1]
    np_rows = (n + 16 * CHUNK - 1) // (16 * CHUNK) * (16 * CHUNK)
    br = 1024
    grid = (np_rows // br,)

    ep_w = ((e + NW - 1) // NW + CHUNK - 1) // CHUNK * CHUNK
    e_pad = ep_w * NW
    pad = e_pad - e
    pad_i = jnp.arange(pad, dtype=jnp.int32)
    srcp = jnp.concatenate([edge_index[0], n + (pad_i % 8)])
    dstp = jnp.concatenate([edge_index[1], n + 8 + (pad_i % 32)])
    xpad = _pad2(x, np_rows, f_in)
    eps_p = _pad2(eps, np_rows, k_out)
    W2p = _pad2(W2, W2.shape[0], d2p)
    W3p = _pad2(W3, d2p, d3p)
    W4p = _pad2(W4, d3p, d4p)
    W_mup = _pad2(W_mu, d4p, k_out)
    W_lsp = _pad2(W_ls, d4p, k_out)
    b1r = b1[None, :]
    b2p = _pad2(b2[None, :], 1, d2p)
    b3p = _pad2(b3[None, :], 1, d3p)
    b4p = _pad2(b4[None, :], 1, d4p)
    ones = jnp.ones((CHUNK, 8), dtype=jnp.float32)
    n_chunks = ep_w // CHUNK

    row_spec = lambda w: pl.BlockSpec((br, w), lambda i: (i, 0))
    parts_spec = lambda w: pl.BlockSpec((2, br, w), lambda i: (0, i, 0))
    full_spec = lambda a: pl.BlockSpec(a.shape, lambda i: (0,) * a.ndim)
    out_row = lambda w: jax.ShapeDtypeStruct((np_rows, w), jnp.float32)

    degp = _make_deg(np_rows, 8, ep_w, n_chunks)(ones, dstp)

    def tc0(degp_r, x_r, dinv_r, xs_r):
        i = pl.program_id(0)
        deg = degp_r[0, :, 0:1] + degp_r[1, :, 0:1] - 1.0
        rows = i * br + lax.broadcasted_iota(jnp.int32, (br, 1), 0)
        dinv = jnp.where(rows < n, lax.rsqrt(deg), 0.0)
        dinv_r[...] = dinv
        xs_r[...] = x_r[...] * dinv

    dinv, xs1 = pl.pallas_call(
        tc0, grid=grid,
        in_specs=[parts_spec(8), row_spec(f_in)],
        out_specs=[row_spec(1), row_spec(f_in)],
        out_shape=[out_row(1), out_row(f_in)],
    )(degp, xpad)

    p1 = _make_prop(np_rows, f_in, ep_w, n_chunks)(xs1, srcp, dstp)

    def tc1(dinv_r, p_r, xs_r, W1_r, b1_r, W2_r, va_r, vb_r):
        dinv = dinv_r[...]
        u = (p_r[0] + p_r[1] - xs_r[...]) * dinv
        h = jnp.maximum(
            jnp.dot(u, W1_r[...], preferred_element_type=jnp.float32)
            + b1_r[...], 0.0)
        v2 = jnp.dot(h, W2_r[...], preferred_element_type=jnp.float32) * dinv
        va_r[...] = v2[:, :d2a]
        vb_r[...] = v2[:, d2a:]

    v2a, v2b = pl.pallas_call(
        tc1, grid=grid,
        in_specs=[row_spec(1), parts_spec(f_in), row_spec(f_in),
                  full_spec(W1), full_spec(b1r), full_spec(W2p)],
        out_specs=[row_spec(d2a), row_spec(d2b)],
        out_shape=[out_row(d2a), out_row(d2b)],
    )(dinv, p1, xs1, W1, b1r, W2p)

    p2a = _make_prop(np_rows, d2a, ep_w, n_chunks)(v2a, srcp, dstp)
    p2b = _make_prop(np_rows, d2b, ep_w, n_chunks)(v2b, srcp, dstp)

    def tc2(dinv_r, pa_r, pb_r, va_r, vb_r, b_rr, W_rr, vo_r):
        dinv = dinv_r[...]
        s = jnp.concatenate(
            [pa_r[0] + pa_r[1] - va_r[...], pb_r[0] + pb_r[1] - vb_r[...]],
            axis=1)
        h = jnp.maximum(s * dinv + b_rr[...], 0.0)
        vo_r[...] = jnp.dot(
            h, W_rr[...], preferred_element_type=jnp.float32) * dinv

    v3 = pl.pallas_call(
        tc2, grid=grid,
        in_specs=[row_spec(1), parts_spec(d2a), parts_spec(d2b),
                  row_spec(d2a), row_spec(d2b),
                  full_spec(b2p), full_spec(W3p)],
        out_specs=row_spec(d3p),
        out_shape=out_row(d3p),
    )(dinv, p2a, p2b, v2a, v2b, b2p, W3p)

    def mid(v, b_r, W_r, w_in, w_out):
        p = _make_prop(np_rows, w_in, ep_w, n_chunks)(v, srcp, dstp)

        def tc(dinv_r, p_r, v_r, b_rr, W_rr, vo_r):
            dinv = dinv_r[...]
            h = jnp.maximum((p_r[0] + p_r[1] - v_r[...]) * dinv + b_rr[...],
                            0.0)
            vo_r[...] = jnp.dot(
                h, W_rr[...], preferred_element_type=jnp.float32) * dinv

        return pl.pallas_call(
            tc, grid=grid,
            in_specs=[row_spec(1), parts_spec(w_in), row_spec(w_in),
                      full_spec(b_r), full_spec(W_r)],
            out_specs=row_spec(w_out),
            out_shape=out_row(w_out),
        )(dinv, p, v, b_r, W_r)

    v4 = mid(v3, b3p, W4p, d3p, d4p)

    p4 = _make_prop(np_rows, d4p, ep_w, n_chunks)(v4, srcp, dstp)

    def tc4(dinv_r, p_r, v_r, b_rr, v5_r):
        dinv = dinv_r[...]
        h = jnp.maximum((p_r[0] + p_r[1] - v_r[...]) * dinv + b_rr[...], 0.0)
        v5_r[...] = h * dinv

    v5 = pl.pallas_call(
        tc4, grid=grid,
        in_specs=[row_spec(1), parts_spec(d4p), row_spec(d4p),
                  full_spec(b4p)],
        out_specs=row_spec(d4p),
        out_shape=out_row(d4p),
    )(dinv, p4, v4, b4p)

    p5 = _make_prop(np_rows, d4p, ep_w, n_chunks)(v5, srcp, dstp)

    def tc5(dinv_r, p_r, v_r, Wm_r, bm_r, Wl_r, bl_r, eps_r, pz_r, z_r):
        g = (p_r[0] + p_r[1] - v_r[...]) * dinv_r[...]
        mu = jnp.dot(g, Wm_r[...], preferred_element_type=jnp.float32) \
            + bm_r[...]
        ls = jnp.dot(g, Wl_r[...], preferred_element_type=jnp.float32) \
            + bl_r[...]
        z = mu + eps_r[...] * jnp.exp(ls)
        m = jnp.max(z, axis=1, keepdims=True)
        lse = m + jnp.log(jnp.sum(jnp.exp(z - m), axis=1, keepdims=True))
        pz_r[...] = z - lse
        z_r[...] = z

    pz, z = pl.pallas_call(
        tc5, grid=grid,
        in_specs=[row_spec(1), parts_spec(d4p), row_spec(d4p),
                  full_spec(W_mup), full_spec(b_mu[None, :]),
                  full_spec(W_lsp), full_spec(b_ls[None, :]),
                  row_spec(k_out)],
        out_specs=[row_spec(k_out), row_spec(k_out)],
        out_shape=[out_row(k_out), out_row(k_out)],
    )(dinv, p5, v5, W_mup, b_mu[None, :], W_lsp, b_ls[None, :], eps_p)

    return (pz[:n], z[:n])


# speedup vs baseline: 17.0106x; 1.2239x over previous
import functools

import jax
import jax.numpy as jnp
from jax import lax
from jax.experimental import pallas as pl
from jax.experimental.pallas import tpu as pltpu
from jax.experimental.pallas import tpu_sc as plsc

NC = 2
NS = 16
NW = NC * NS
CHUNK = 128

_MESH = plsc.VectorSubcoreMesh(core_axis_name="c", subcore_axis_name="s")
_SC_PARAMS = pltpu.CompilerParams(use_tc_tiling_on_sc=False)


def _make_prop(np_rows, d, ep_w, gk):
    rpt = np_rows // NS
    n_chunks = ep_w // CHUNK

    @functools.partial(
        pl.kernel,
        out_type=jax.ShapeDtypeStruct((2, np_rows, d), jnp.float32),
        mesh=_MESH,
        scratch_types=(
            [pltpu.VMEM((ep_w,), jnp.int32)] * 2
            + [pltpu.VMEM((CHUNK, d), jnp.float32)] * gk
            + [pltpu.VMEM((CHUNK,), jnp.int32)] * (2 * gk)
            + [pltpu.VMEM_SHARED((np_rows, d), jnp.float32)]
            + [pltpu.SemaphoreType.DMA] * 3
        ),
        compiler_params=_SC_PARAMS,
    )
    def prop(xin, srcp, dstp, out, *refs):
        src_all, dst_all = refs[0], refs[1]
        rows = refs[2:2 + gk]
        sv = refs[2 + gk:2 + 2 * gk]
        dv = refs[2 + 2 * gk:2 + 3 * gk]
        acc = refs[2 + 3 * gk]
        sem_i, sem_g, sem_s = refs[3 + 3 * gk:6 + 3 * gk]
        cid = lax.axis_index("c")
        sid = lax.axis_index("s")
        wid = cid * NS + sid
        r0 = sid * rpt

        ebase = wid * ep_w
        ic0 = pltpu.async_copy(srcp.at[pl.ds(ebase, ep_w)], src_all, sem_i)
        ic1 = pltpu.async_copy(dstp.at[pl.ds(ebase, ep_w)], dst_all, sem_i)
        n_init = rpt // CHUNK
        for i in range(n_init):
            rb = r0 + i * CHUNK
            pltpu.sync_copy(xin.at[pl.ds(rb, CHUNK)], rows[i % 2])
            pltpu.sync_copy(rows[i % 2], acc.at[pl.ds(rb, CHUNK)])
        ic0.wait()
        ic1.wait()
        plsc.subcore_barrier()

        def body(j, carry):
            gd = []
            for k in range(gk):
                cb = (gk * j + k) * CHUNK
                for t in range(CHUNK // 16):
                    sv[k][pl.ds(16 * t, 16)] = src_all[pl.ds(cb + 16 * t, 16)]
                    dv[k][pl.ds(16 * t, 16)] = dst_all[pl.ds(cb + 16 * t, 16)]
                gd.append(pltpu.async_copy(xin.at[sv[k]], rows[k], sem_g))
            sd = []
            for k in range(gk):
                gd[k].wait()
                sd.append(pltpu.async_copy(
                    rows[k], acc.at[dv[k]], sem_s, add=True))
            for k in range(gk):
                sd[k].wait()
            return carry

        lax.fori_loop(0, n_chunks // gk, body, 0)
        plsc.subcore_barrier()

        for i in range(n_init):
            rb = r0 + i * CHUNK
            pltpu.sync_copy(acc.at[pl.ds(rb, CHUNK)], rows[i % 2])
            pltpu.sync_copy(rows[i % 2], out.at[cid, pl.ds(rb, CHUNK)])

    return prop


def _make_deg(np_rows, degw, ep_w, n_chunks):
    rpt = np_rows // NS

    @functools.partial(
        pl.kernel,
        out_type=jax.ShapeDtypeStruct((2, np_rows, degw), jnp.float32),
        mesh=_MESH,
        scratch_types=[
            pltpu.VMEM((ep_w // CHUNK, CHUNK), jnp.int32),
            pltpu.VMEM((CHUNK, degw), jnp.float32),
            pltpu.VMEM_SHARED((np_rows, degw), jnp.float32),
            pltpu.SemaphoreType.DMA,
            pltpu.SemaphoreType.DMA,
        ],
        compiler_params=_SC_PARAMS,
    )
    def deg(ones_hbm, dstp, out, dst_all, ones_v, acc, sem_i, sem_s):
        cid = lax.axis_index("c")
        sid = lax.axis_index("s")
        wid = cid * NS + sid
        r0 = sid * rpt
        ic = pltpu.async_copy(dstp.at[wid], dst_all, sem_i)
        pltpu.sync_copy(ones_hbm, ones_v)

        def init_body(i, carry):
            pltpu.sync_copy(ones_v, acc.at[pl.ds(r0 + i * CHUNK, CHUNK)])
            return carry

        lax.fori_loop(0, rpt // CHUNK, init_body, 0)
        ic.wait()
        plsc.subcore_barrier()

        def edge_body(i, carry):
            sd = []
            for k in range(8):
                sd.append(pltpu.async_copy(
                    ones_v, acc.at[dst_all.at[8 * i + k]], sem_s, add=True))
            for k in range(8):
                sd[k].wait()
            return carry

        lax.fori_loop(0, n_chunks // 8, edge_body, 0)
        plsc.subcore_barrier()

        def out_body(i, carry):
            rb = r0 + i * CHUNK
            pltpu.sync_copy(acc.at[pl.ds(rb, CHUNK)], ones_v)
            pltpu.sync_copy(ones_v, out.at[cid, pl.ds(rb, CHUNK)])
            return carry

        lax.fori_loop(0, rpt // CHUNK, out_body, 0)

    return deg


def _pad2(a, rows, cols):
    return jnp.pad(a, ((0, rows - a.shape[0]), (0, cols - a.shape[1])))


def kernel(x, edge_index, eps, W1, b1, W2, b2, W3, b3, W4, b4,
           W_mu, b_mu, W_ls, b_ls):
    n = x.shape[0]
    e = edge_index.shape[1]
    f_in = x.shape[1]
    k_out = W_mu.shape[1]
    np_rows = (n + 16 * CHUNK - 1) // (16 * CHUNK) * (16 * CHUNK)
    br = 1024
    grid = (np_rows // br,)

    gchunk = 40 * CHUNK
    ep_w = ((e + NW - 1) // NW + gchunk - 1) // gchunk * gchunk
    e_pad = ep_w * NW
    pad = e_pad - e
    pad_i = jnp.arange(pad, dtype=jnp.int32)
    srcp = jnp.concatenate([edge_index[0], n + (pad_i % 8)])
    dstp = jnp.concatenate([edge_index[1], n + 8 + (pad_i % 32)])
    n_chunks = ep_w // CHUNK
    dstp3 = dstp.reshape(NW, n_chunks, CHUNK)
    xpad = _pad2(x, np_rows, f_in)
    eps_p = _pad2(eps, np_rows, k_out)
    W2p = _pad2(W2, W2.shape[0], 176)
    W3p = _pad2(W3, 176, 96)
    W4p = _pad2(W4, 96, 48)
    W_mup = _pad2(W_mu, 48, k_out)
    W_lsp = _pad2(W_ls, 48, k_out)
    b1r = b1[None, :]
    b2p = _pad2(b2[None, :], 1, 176)
    b3p = _pad2(b3[None, :], 1, 96)
    b4p = _pad2(b4[None, :], 1, 48)
    ones = jnp.ones((CHUNK, 8), dtype=jnp.float32)

    row_spec = lambda w: pl.BlockSpec((br, w), lambda i: (i, 0))
    parts_spec = lambda w: pl.BlockSpec((2, br, w), lambda i: (0, i, 0))
    full_spec = lambda a: pl.BlockSpec(a.shape, lambda i: (0,) * a.ndim)
    out_row = lambda w: jax.ShapeDtypeStruct((np_rows, w), jnp.float32)

    prop64 = _make_prop(np_rows, 64, ep_w, 5)
    prop48 = _make_prop(np_rows, 48, ep_w, 10)

    def prop_sum(parts, vs):
        return jnp.concatenate(
            [p[0] + p[1] - v[...] for p, v in zip(parts, vs)], axis=1)

    degp = _make_deg(np_rows, 8, ep_w, n_chunks)(ones, dstp3)

    def tc0(degp_r, x_r, dinv_r, xsa_r, xsb_r):
        i = pl.program_id(0)
        deg = degp_r[0, :, 0:1] + degp_r[1, :, 0:1] - 1.0
        rows = i * br + lax.broadcasted_iota(jnp.int32, (br, 1), 0)
        dinv = jnp.where(rows < n, lax.rsqrt(deg), 0.0)
        dinv_r[...] = dinv
        xs = x_r[...] * dinv
        xsa_r[...] = xs[:, :64]
        xsb_r[...] = xs[:, 64:]

    dinv, xs1a, xs1b = pl.pallas_call(
        tc0, grid=grid,
        in_specs=[parts_spec(8), row_spec(f_in)],
        out_specs=[row_spec(1), row_spec(64), row_spec(64)],
        out_shape=[out_row(1), out_row(64), out_row(64)],
    )(degp, xpad)

    p1a = prop64(xs1a, srcp, dstp)
    p1b = prop64(xs1b, srcp, dstp)

    def tc1(dinv_r, pa_r, pb_r, xsa_r, xsb_r, W1_r, b1_r, W2_r,
            va_r, vb_r, vc_r):
        dinv = dinv_r[...]
        u = prop_sum([pa_r, pb_r], [xsa_r, xsb_r]) * dinv
        h = jnp.maximum(
            jnp.dot(u, W1_r[...], preferred_element_type=jnp.float32)
            + b1_r[...], 0.0)
        v2 = jnp.dot(h, W2_r[...], preferred_element_type=jnp.float32) * dinv
        va_r[...] = v2[:, :64]
        vb_r[...] = v2[:, 64:128]
        vc_r[...] = v2[:, 128:]

    v2a, v2b, v2c = pl.pallas_call(
        tc1, grid=grid,
        in_specs=[row_spec(1), parts_spec(64), parts_spec(64),
                  row_spec(64), row_spec(64),
                  full_spec(W1), full_spec(b1r), full_spec(W2p)],
        out_specs=[row_spec(64), row_spec(64), row_spec(48)],
        out_shape=[out_row(64), out_row(64), out_row(48)],
    )(dinv, p1a, p1b, xs1a, xs1b, W1, b1r, W2p)

    p2a = prop64(v2a, srcp, dstp)
    p2b = prop64(v2b, srcp, dstp)
    p2c = prop48(v2c, srcp, dstp)

    def tc2(dinv_r, pa_r, pb_r, pc_r, va_r, vb_r, vc_r, b_rr, W_rr,
            voa_r, vob_r):
        dinv = dinv_r[...]
        s = prop_sum([pa_r, pb_r, pc_r], [va_r, vb_r, vc_r])
        h = jnp.maximum(s * dinv + b_rr[...], 0.0)
        v3 = jnp.dot(h, W_rr[...], preferred_element_type=jnp.float32) * dinv
        voa_r[...] = v3[:, :48]
        vob_r[...] = v3[:, 48:]

    v3a, v3b = pl.pallas_call(
        tc2, grid=grid,
        in_specs=[row_spec(1), parts_spec(64), parts_spec(64), parts_spec(48),
                  row_spec(64), row_spec(64), row_spec(48),
                  full_spec(b2p), full_spec(W3p)],
        out_specs=[row_spec(48), row_spec(48)],
        out_shape=[out_row(48), out_row(48)],
    )(dinv, p2a, p2b, p2c, v2a, v2b, v2c, b2p, W3p)

    p3a = prop48(v3a, srcp, dstp)
    p3b = prop48(v3b, srcp, dstp)

    def tc3(dinv_r, pa_r, pb_r, va_r, vb_r, b_rr, W_rr, vo_r):
        dinv = dinv_r[...]
        s = prop_sum([pa_r, pb_r], [va_r, vb_r])
        h = jnp.maximum(s * dinv + b_rr[...], 0.0)
        vo_r[...] = jnp.dot(
            h, W_rr[...], preferred_element_type=jnp.float32) * dinv

    v4 = pl.pallas_call(
        tc3, grid=grid,
        in_specs=[row_spec(1), parts_spec(48), parts_spec(48),
                  row_spec(48), row_spec(48),
                  full_spec(b3p), full_spec(W4p)],
        out_specs=row_spec(48),
        out_shape=out_row(48),
    )(dinv, p3a, p3b, v3a, v3b, b3p, W4p)

    p4 = prop48(v4, srcp, dstp)

    def tc4(dinv_r, p_r, v_r, b_rr, v5_r):
        dinv = dinv_r[...]
        h = jnp.maximum((p_r[0] + p_r[1] - v_r[...]) * dinv + b_rr[...], 0.0)
        v5_r[...] = h * dinv

    v5 = pl.pallas_call(
        tc4, grid=grid,
        in_specs=[row_spec(1), parts_spec(48), row_spec(48),
                  full_spec(b4p)],
        out_specs=row_spec(48),
        out_shape=out_row(48),
    )(dinv, p4, v4, b4p)

    p5 = prop48(v5, srcp, dstp)

    def tc5(dinv_r, p_r, v_r, Wm_r, bm_r, Wl_r, bl_r, eps_r, pz_r, z_r):
        g = (p_r[0] + p_r[1] - v_r[...]) * dinv_r[...]
        mu = jnp.dot(g, Wm_r[...], preferred_element_type=jnp.float32) \
            + bm_r[...]
        ls = jnp.dot(g, Wl_r[...], preferred_element_type=jnp.float32) \
            + bl_r[...]
        z = mu + eps_r[...] * jnp.exp(ls)
        m = jnp.max(z, axis=1, keepdims=True)
        lse = m + jnp.log(jnp.sum(jnp.exp(z - m), axis=1, keepdims=True))
        pz_r[...] = z - lse
        z_r[...] = z

    pz, z = pl.pallas_call(
        tc5, grid=grid,
        in_specs=[row_spec(1), parts_spec(48), row_spec(48),
                  full_spec(W_mup), full_spec(b_mu[None, :]),
                  full_spec(W_lsp), full_spec(b_ls[None, :]),
                  row_spec(k_out)],
        out_specs=[row_spec(k_out), row_spec(k_out)],
        out_shape=[out_row(k_out), out_row(k_out)],
    )(dinv, p5, v5, W_mup, b_mu[None, :], W_lsp, b_ls[None, :], eps_p)

    return (pz[:n], z[:n])


# speedup vs baseline: 17.2934x; 1.0166x over previous
import functools

import jax
import jax.numpy as jnp
from jax import lax
from jax.experimental import pallas as pl
from jax.experimental.pallas import tpu as pltpu
from jax.experimental.pallas import tpu_sc as plsc

NC = 2
NS = 16
NW = NC * NS
CHUNK = 128

_MESH = plsc.VectorSubcoreMesh(core_axis_name="c", subcore_axis_name="s")
_SC_PARAMS = pltpu.CompilerParams(use_tc_tiling_on_sc=False)


def _make_prop(np_rows, d, ep_w, gk):
    rpt = np_rows // NS
    n_chunks = ep_w // CHUNK

    @functools.partial(
        pl.kernel,
        out_type=jax.ShapeDtypeStruct((2, np_rows, d), jnp.float32),
        mesh=_MESH,
        scratch_types=(
            [pltpu.VMEM((ep_w,), jnp.int32)] * 2
            + [pltpu.VMEM((CHUNK, d), jnp.float32)] * gk
            + [pltpu.VMEM((CHUNK,), jnp.int32)] * (2 * gk)
            + [pltpu.VMEM_SHARED((np_rows, d), jnp.float32)]
            + [pltpu.SemaphoreType.DMA] * 3
        ),
        compiler_params=_SC_PARAMS,
    )
    def prop(xin, srcp, dstp, out, *refs):
        src_all, dst_all = refs[0], refs[1]
        rows = refs[2:2 + gk]
        sv = refs[2 + gk:2 + 2 * gk]
        dv = refs[2 + 2 * gk:2 + 3 * gk]
        acc = refs[2 + 3 * gk]
        sem_i, sem_g, sem_s = refs[3 + 3 * gk:6 + 3 * gk]
        cid = lax.axis_index("c")
        sid = lax.axis_index("s")
        wid = cid * NS + sid
        r0 = sid * rpt

        ebase = wid * ep_w
        ic0 = pltpu.async_copy(srcp.at[pl.ds(ebase, ep_w)], src_all, sem_i)
        ic1 = pltpu.async_copy(dstp.at[pl.ds(ebase, ep_w)], dst_all, sem_i)
        n_init = rpt // CHUNK
        for i in range(n_init):
            rb = r0 + i * CHUNK
            pltpu.sync_copy(xin.at[pl.ds(rb, CHUNK)], rows[i % 2])
            pltpu.sync_copy(rows[i % 2], acc.at[pl.ds(rb, CHUNK)])
        ic0.wait()
        ic1.wait()
        plsc.subcore_barrier()

        def body(j, carry):
            gd = []
            for k in range(gk):
                cb = (gk * j + k) * CHUNK

                @pl.when(j >= 1)
                def _():
                    pltpu.make_async_copy(
                        rows[k], acc.at[dv[k]], sem_s).wait()

                for t in range(CHUNK // 16):
                    sv[k][pl.ds(16 * t, 16)] = src_all[pl.ds(cb + 16 * t, 16)]
                    dv[k][pl.ds(16 * t, 16)] = dst_all[pl.ds(cb + 16 * t, 16)]
                gd.append(pltpu.async_copy(xin.at[sv[k]], rows[k], sem_g))
            for k in range(gk):
                gd[k].wait()
                pltpu.async_copy(rows[k], acc.at[dv[k]], sem_s, add=True)
            return carry

        lax.fori_loop(0, n_chunks // gk, body, 0)
        for k in range(gk):
            pltpu.make_async_copy(rows[k], acc.at[dv[k]], sem_s).wait()
        plsc.subcore_barrier()

        for i in range(n_init):
            rb = r0 + i * CHUNK
            pltpu.sync_copy(acc.at[pl.ds(rb, CHUNK)], rows[i % 2])
            pltpu.sync_copy(rows[i % 2], out.at[cid, pl.ds(rb, CHUNK)])

    return prop


def _make_deg(np_rows, degw, ep_w, n_chunks):
    rpt = np_rows // NS

    @functools.partial(
        pl.kernel,
        out_type=jax.ShapeDtypeStruct((2, np_rows, degw), jnp.float32),
        mesh=_MESH,
        scratch_types=[
            pltpu.VMEM((ep_w // CHUNK, CHUNK), jnp.int32),
            pltpu.VMEM((CHUNK, degw), jnp.float32),
            pltpu.VMEM_SHARED((np_rows, degw), jnp.float32),
            pltpu.SemaphoreType.DMA,
            pltpu.SemaphoreType.DMA,
        ],
        compiler_params=_SC_PARAMS,
    )
    def deg(ones_hbm, dstp, out, dst_all, ones_v, acc, sem_i, sem_s):
        cid = lax.axis_index("c")
        sid = lax.axis_index("s")
        wid = cid * NS + sid
        r0 = sid * rpt
        ic = pltpu.async_copy(dstp.at[wid], dst_all, sem_i)
        pltpu.sync_copy(ones_hbm, ones_v)

        def init_body(i, carry):
            pltpu.sync_copy(ones_v, acc.at[pl.ds(r0 + i * CHUNK, CHUNK)])
            return carry

        lax.fori_loop(0, rpt // CHUNK, init_body, 0)
        ic.wait()
        plsc.subcore_barrier()

        def edge_body(i, carry):
            sd = []
            for k in range(8):
                sd.append(pltpu.async_copy(
                    ones_v, acc.at[dst_all.at[8 * i + k]], sem_s, add=True))
            for k in range(8):
                sd[k].wait()
            return carry

        lax.fori_loop(0, n_chunks // 8, edge_body, 0)
        plsc.subcore_barrier()

        def out_body(i, carry):
            rb = r0 + i * CHUNK
            pltpu.sync_copy(acc.at[pl.ds(rb, CHUNK)], ones_v)
            pltpu.sync_copy(ones_v, out.at[cid, pl.ds(rb, CHUNK)])
            return carry

        lax.fori_loop(0, rpt // CHUNK, out_body, 0)

    return deg


def _pad2(a, rows, cols):
    return jnp.pad(a, ((0, rows - a.shape[0]), (0, cols - a.shape[1])))


def kernel(x, edge_index, eps, W1, b1, W2, b2, W3, b3, W4, b4,
           W_mu, b_mu, W_ls, b_ls):
    n = x.shape[0]
    e = edge_index.shape[1]
    f_in = x.shape[1]
    k_out = W_mu.shape[1]
    np_rows = (n + 16 * CHUNK - 1) // (16 * CHUNK) * (16 * CHUNK)
    br = 1024
    grid = (np_rows // br,)

    gchunk = 40 * CHUNK
    ep_w = ((e + NW - 1) // NW + gchunk - 1) // gchunk * gchunk
    e_pad = ep_w * NW
    pad = e_pad - e
    pad_i = jnp.arange(pad, dtype=jnp.int32)
    srcp = jnp.concatenate([edge_index[0], n + (pad_i % 8)])
    dstp = jnp.concatenate([edge_index[1], n + 8 + (pad_i % 32)])
    n_chunks = ep_w // CHUNK
    dstp3 = dstp.reshape(NW, n_chunks, CHUNK)
    xpad = _pad2(x, np_rows, f_in)
    eps_p = _pad2(eps, np_rows, k_out)
    W2p = _pad2(W2, W2.shape[0], 176)
    W3p = _pad2(W3, 176, 96)
    W4p = _pad2(W4, 96, 48)
    W_mup = _pad2(W_mu, 48, k_out)
    W_lsp = _pad2(W_ls, 48, k_out)
    b1r = b1[None, :]
    b2p = _pad2(b2[None, :], 1, 176)
    b3p = _pad2(b3[None, :], 1, 96)
    b4p = _pad2(b4[None, :], 1, 48)
    ones = jnp.ones((CHUNK, 8), dtype=jnp.float32)

    row_spec = lambda w: pl.BlockSpec((br, w), lambda i: (i, 0))
    parts_spec = lambda w: pl.BlockSpec((2, br, w), lambda i: (0, i, 0))
    full_spec = lambda a: pl.BlockSpec(a.shape, lambda i: (0,) * a.ndim)
    out_row = lambda w: jax.ShapeDtypeStruct((np_rows, w), jnp.float32)

    prop64 = _make_prop(np_rows, 64, ep_w, 5)
    prop48 = _make_prop(np_rows, 48, ep_w, 10)

    def prop_sum(parts, vs):
        return jnp.concatenate(
            [p[0] + p[1] - v[...] for p, v in zip(parts, vs)], axis=1)

    degp = _make_deg(np_rows, 8, ep_w, n_chunks)(ones, dstp3)

    def tc0(degp_r, x_r, dinv_r, xsa_r, xsb_r):
        i = pl.program_id(0)
        deg = degp_r[0, :, 0:1] + degp_r[1, :, 0:1] - 1.0
        rows = i * br + lax.broadcasted_iota(jnp.int32, (br, 1), 0)
        dinv = jnp.where(rows < n, lax.rsqrt(deg), 0.0)
        dinv_r[...] = dinv
        xs = x_r[...] * dinv
        xsa_r[...] = xs[:, :64]
        xsb_r[...] = xs[:, 64:]

    dinv, xs1a, xs1b = pl.pallas_call(
        tc0, grid=grid,
        in_specs=[parts_spec(8), row_spec(f_in)],
        out_specs=[row_spec(1), row_spec(64), row_spec(64)],
        out_shape=[out_row(1), out_row(64), out_row(64)],
    )(degp, xpad)

    p1a = prop64(xs1a, srcp, dstp)
    p1b = prop64(xs1b, srcp, dstp)

    def tc1(dinv_r, pa_r, pb_r, xsa_r, xsb_r, W1_r, b1_r, W2_r,
            va_r, vb_r, vc_r):
        dinv = dinv_r[...]
        u = prop_sum([pa_r, pb_r], [xsa_r, xsb_r]) * dinv
        h = jnp.maximum(
            jnp.dot(u, W1_r[...], preferred_element_type=jnp.float32)
            + b1_r[...], 0.0)
        v2 = jnp.dot(h, W2_r[...], preferred_element_type=jnp.float32) * dinv
        va_r[...] = v2[:, :64]
        vb_r[...] = v2[:, 64:128]
        vc_r[...] = v2[:, 128:]

    v2a, v2b, v2c = pl.pallas_call(
        tc1, grid=grid,
        in_specs=[row_spec(1), parts_spec(64), parts_spec(64),
                  row_spec(64), row_spec(64),
                  full_spec(W1), full_spec(b1r), full_spec(W2p)],
        out_specs=[row_spec(64), row_spec(64), row_spec(48)],
        out_shape=[out_row(64), out_row(64), out_row(48)],
    )(dinv, p1a, p1b, xs1a, xs1b, W1, b1r, W2p)

    p2a = prop64(v2a, srcp, dstp)
    p2b = prop64(v2b, srcp, dstp)
    p2c = prop48(v2c, srcp, dstp)

    def tc2(dinv_r, pa_r, pb_r, pc_r, va_r, vb_r, vc_r, b_rr, W_rr,
            voa_r, vob_r):
        dinv = dinv_r[...]
        s = prop_sum([pa_r, pb_r, pc_r], [va_r, vb_r, vc_r])
        h = jnp.maximum(s * dinv + b_rr[...], 0.0)
        v3 = jnp.dot(h, W_rr[...], preferred_element_type=jnp.float32) * dinv
        voa_r[...] = v3[:, :48]
        vob_r[...] = v3[:, 48:]

    v3a, v3b = pl.pallas_call(
        tc2, grid=grid,
        in_specs=[row_spec(1), parts_spec(64), parts_spec(64), parts_spec(48),
                  row_spec(64), row_spec(64), row_spec(48),
                  full_spec(b2p), full_spec(W3p)],
        out_specs=[row_spec(48), row_spec(48)],
        out_shape=[out_row(48), out_row(48)],
    )(dinv, p2a, p2b, p2c, v2a, v2b, v2c, b2p, W3p)

    p3a = prop48(v3a, srcp, dstp)
    p3b = prop48(v3b, srcp, dstp)

    def tc3(dinv_r, pa_r, pb_r, va_r, vb_r, b_rr, W_rr, vo_r):
        dinv = dinv_r[...]
        s = prop_sum([pa_r, pb_r], [va_r, vb_r])
        h = jnp.maximum(s * dinv + b_rr[...], 0.0)
        vo_r[...] = jnp.dot(
            h, W_rr[...], preferred_element_type=jnp.float32) * dinv

    v4 = pl.pallas_call(
        tc3, grid=grid,
        in_specs=[row_spec(1), parts_spec(48), parts_spec(48),
                  row_spec(48), row_spec(48),
                  full_spec(b3p), full_spec(W4p)],
        out_specs=row_spec(48),
        out_shape=out_row(48),
    )(dinv, p3a, p3b, v3a, v3b, b3p, W4p)

    p4 = prop48(v4, srcp, dstp)

    def tc4(dinv_r, p_r, v_r, b_rr, v5_r):
        dinv = dinv_r[...]
        h = jnp.maximum((p_r[0] + p_r[1] - v_r[...]) * dinv + b_rr[...], 0.0)
        v5_r[...] = h * dinv

    v5 = pl.pallas_call(
        tc4, grid=grid,
        in_specs=[row_spec(1), parts_spec(48), row_spec(48),
                  full_spec(b4p)],
        out_specs=row_spec(48),
        out_shape=out_row(48),
    )(dinv, p4, v4, b4p)

    p5 = prop48(v5, srcp, dstp)

    def tc5(dinv_r, p_r, v_r, Wm_r, bm_r, Wl_r, bl_r, eps_r, pz_r, z_r):
        g = (p_r[0] + p_r[1] - v_r[...]) * dinv_r[...]
        mu = jnp.dot(g, Wm_r[...], preferred_element_type=jnp.float32) \
            + bm_r[...]
        ls = jnp.dot(g, Wl_r[...], preferred_element_type=jnp.float32) \
            + bl_r[...]
        z = mu + eps_r[...] * jnp.exp(ls)
        m = jnp.max(z, axis=1, keepdims=True)
        lse = m + jnp.log(jnp.sum(jnp.exp(z - m), axis=1, keepdims=True))
        pz_r[...] = z - lse
        z_r[...] = z

    pz, z = pl.pallas_call(
        tc5, grid=grid,
        in_specs=[row_spec(1), parts_spec(48), row_spec(48),
                  full_spec(W_mup), full_spec(b_mu[None, :]),
                  full_spec(W_lsp), full_spec(b_ls[None, :]),
                  row_spec(k_out)],
        out_specs=[row_spec(k_out), row_spec(k_out)],
        out_shape=[out_row(k_out), out_row(k_out)],
    )(dinv, p5, v5, W_mup, b_mu[None, :], W_lsp, b_ls[None, :], eps_p)

    return (pz[:n], z[:n])


# speedup vs baseline: 20.9801x; 1.2132x over previous
import functools

import jax
import jax.numpy as jnp
from jax import lax
from jax.experimental import pallas as pl
from jax.experimental.pallas import tpu as pltpu
from jax.experimental.pallas import tpu_sc as plsc

NC = 2
NS = 16
NW = NC * NS
CHUNK = 128

_MESH = plsc.VectorSubcoreMesh(core_axis_name="c", subcore_axis_name="s")
_SC_PARAMS = pltpu.CompilerParams(use_tc_tiling_on_sc=False)


def _make_prop(np_rows, d, ep_w, gk, chunk=CHUNK):
    rpt = np_rows // NS
    n_chunks = ep_w // chunk

    @functools.partial(
        pl.kernel,
        out_type=jax.ShapeDtypeStruct((2, np_rows, d), jnp.float32),
        mesh=_MESH,
        scratch_types=(
            [pltpu.VMEM((ep_w,), jnp.int32)] * 2
            + [pltpu.VMEM((chunk, d), jnp.float32)] * gk
            + [pltpu.VMEM((chunk,), jnp.int32)] * (2 * gk)
            + [pltpu.VMEM_SHARED((np_rows, d), jnp.float32)]
            + [pltpu.SemaphoreType.DMA] * 3
        ),
        compiler_params=_SC_PARAMS,
    )
    def prop(xin, srcp, dstp, out, *refs):
        src_all, dst_all = refs[0], refs[1]
        rows = refs[2:2 + gk]
        sv = refs[2 + gk:2 + 2 * gk]
        dv = refs[2 + 2 * gk:2 + 3 * gk]
        acc = refs[2 + 3 * gk]
        sem_i, sem_g, sem_s = refs[3 + 3 * gk:6 + 3 * gk]
        cid = lax.axis_index("c")
        sid = lax.axis_index("s")
        wid = cid * NS + sid
        r0 = sid * rpt

        ebase = wid * ep_w
        ic0 = pltpu.async_copy(srcp.at[pl.ds(ebase, ep_w)], src_all, sem_i)
        ic1 = pltpu.async_copy(dstp.at[pl.ds(ebase, ep_w)], dst_all, sem_i)
        n_init = rpt // chunk
        for i in range(n_init):
            rb = r0 + i * chunk
            pltpu.sync_copy(xin.at[pl.ds(rb, chunk)], rows[i % 2])
            pltpu.sync_copy(rows[i % 2], acc.at[pl.ds(rb, chunk)])
        ic0.wait()
        ic1.wait()
        plsc.subcore_barrier()

        def body(j, carry):
            gd = []
            for k in range(gk):
                cb = (gk * j + k) * chunk

                @pl.when(j >= 1)
                def _():
                    pltpu.make_async_copy(
                        rows[k], acc.at[dv[k]], sem_s).wait()

                for t in range(chunk // 16):
                    sv[k][pl.ds(16 * t, 16)] = src_all[pl.ds(cb + 16 * t, 16)]
                    dv[k][pl.ds(16 * t, 16)] = dst_all[pl.ds(cb + 16 * t, 16)]
                gd.append(pltpu.async_copy(xin.at[sv[k]], rows[k], sem_g))
            for k in range(gk):
                gd[k].wait()
                pltpu.async_copy(rows[k], acc.at[dv[k]], sem_s, add=True)
            return carry

        lax.fori_loop(0, n_chunks // gk, body, 0)
        for k in range(gk):
            pltpu.make_async_copy(rows[k], acc.at[dv[k]], sem_s).wait()
        plsc.subcore_barrier()

        for i in range(n_init):
            rb = r0 + i * chunk
            pltpu.sync_copy(acc.at[pl.ds(rb, chunk)], rows[i % 2])
            pltpu.sync_copy(rows[i % 2], out.at[cid, pl.ds(rb, chunk)])

    return prop


def _make_deg(np_rows, degw, ep_w, n_chunks):
    rpt = np_rows // NS

    @functools.partial(
        pl.kernel,
        out_type=jax.ShapeDtypeStruct((2, np_rows, degw), jnp.float32),
        mesh=_MESH,
        scratch_types=[
            pltpu.VMEM((ep_w // CHUNK, CHUNK), jnp.int32),
            pltpu.VMEM((CHUNK, degw), jnp.float32),
            pltpu.VMEM_SHARED((np_rows, degw), jnp.float32),
            pltpu.SemaphoreType.DMA,
            pltpu.SemaphoreType.DMA,
        ],
        compiler_params=_SC_PARAMS,
    )
    def deg(ones_hbm, dstp, out, dst_all, ones_v, acc, sem_i, sem_s):
        cid = lax.axis_index("c")
        sid = lax.axis_index("s")
        wid = cid * NS + sid
        r0 = sid * rpt
        ic = pltpu.async_copy(dstp.at[wid], dst_all, sem_i)
        pltpu.sync_copy(ones_hbm, ones_v)

        def init_body(i, carry):
            pltpu.sync_copy(ones_v, acc.at[pl.ds(r0 + i * CHUNK, CHUNK)])
            return carry

        lax.fori_loop(0, rpt // CHUNK, init_body, 0)
        ic.wait()
        plsc.subcore_barrier()

        def edge_body(i, carry):
            sd = []
            for k in range(8):
                sd.append(pltpu.async_copy(
                    ones_v, acc.at[dst_all.at[8 * i + k]], sem_s, add=True))
            for k in range(8):
                sd[k].wait()
            return carry

        lax.fori_loop(0, n_chunks // 8, edge_body, 0)
        plsc.subcore_barrier()

        def out_body(i, carry):
            rb = r0 + i * CHUNK
            pltpu.sync_copy(acc.at[pl.ds(rb, CHUNK)], ones_v)
            pltpu.sync_copy(ones_v, out.at[cid, pl.ds(rb, CHUNK)])
            return carry

        lax.fori_loop(0, rpt // CHUNK, out_body, 0)

    return deg


def _pad2(a, rows, cols):
    return jnp.pad(a, ((0, rows - a.shape[0]), (0, cols - a.shape[1])))


def kernel(x, edge_index, eps, W1, b1, W2, b2, W3, b3, W4, b4,
           W_mu, b_mu, W_ls, b_ls):
    n = x.shape[0]
    e = edge_index.shape[1]
    f_in = x.shape[1]
    k_out = W_mu.shape[1]
    np_rows = (n + 16 * CHUNK - 1) // (16 * CHUNK) * (16 * CHUNK)
    br = 1024
    grid = (np_rows // br,)

    gchunk = 40 * CHUNK
    ep_w = ((e + NW - 1) // NW + gchunk - 1) // gchunk * gchunk
    e_pad = ep_w * NW
    pad = e_pad - e
    pad_i = jnp.arange(pad, dtype=jnp.int32)
    srcp = jnp.concatenate([edge_index[0], n + (pad_i % 8)])
    dstp = jnp.concatenate([edge_index[1], n + 8 + (pad_i % 32)])
    n_chunks = ep_w // CHUNK
    dstp3 = dstp.reshape(NW, n_chunks, CHUNK)
    xpad = _pad2(x, np_rows, f_in)
    eps_p = _pad2(eps, np_rows, k_out)
    W2p = _pad2(W2, W2.shape[0], 176)
    W3p = _pad2(W3, 176, 96)
    W4p = _pad2(W4, 96, 48)
    W_mup = _pad2(W_mu, 48, k_out)
    W_lsp = _pad2(W_ls, 48, k_out)
    b1r = b1[None, :]
    b2p = _pad2(b2[None, :], 1, 176)
    b3p = _pad2(b3[None, :], 1, 96)
    b4p = _pad2(b4[None, :], 1, 48)
    ones = jnp.ones((CHUNK, 8), dtype=jnp.float32)

    row_spec = lambda w: pl.BlockSpec((br, w), lambda i: (i, 0))
    parts_spec = lambda w: pl.BlockSpec((2, br, w), lambda i: (0, i, 0))
    full_spec = lambda a: pl.BlockSpec(a.shape, lambda i: (0,) * a.ndim)
    out_row = lambda w: jax.ShapeDtypeStruct((np_rows, w), jnp.float32)

    prop64 = _make_prop(np_rows, 64, ep_w, 5)
    prop48 = _make_prop(np_rows, 48, ep_w, 10)
    prop80 = _make_prop(np_rows, 80, ep_w, 4)
    prop96 = _make_prop(np_rows, 96, ep_w, 5, chunk=64)

    def prop_sum(parts, vs):
        return jnp.concatenate(
            [p[0] + p[1] - v[...] for p, v in zip(parts, vs)], axis=1)

    degp = _make_deg(np_rows, 8, ep_w, n_chunks)(ones, dstp3)

    def tc0(degp_r, x_r, dinv_r, xsa_r, xsb_r):
        i = pl.program_id(0)
        deg = degp_r[0, :, 0:1] + degp_r[1, :, 0:1] - 1.0
        rows = i * br + lax.broadcasted_iota(jnp.int32, (br, 1), 0)
        dinv = jnp.where(rows < n, lax.rsqrt(deg), 0.0)
        dinv_r[...] = dinv
        xs = x_r[...] * dinv
        xsa_r[...] = xs[:, :64]
        xsb_r[...] = xs[:, 64:]

    dinv, xs1a, xs1b = pl.pallas_call(
        tc0, grid=grid,
        in_specs=[parts_spec(8), row_spec(f_in)],
        out_specs=[row_spec(1), row_spec(64), row_spec(64)],
        out_shape=[out_row(1), out_row(64), out_row(64)],
    )(degp, xpad)

    p1a = prop64(xs1a, srcp, dstp)
    p1b = prop64(xs1b, srcp, dstp)

    def tc1(dinv_r, pa_r, pb_r, xsa_r, xsb_r, W1_r, b1_r, W2_r,
            va_r, vb_r):
        dinv = dinv_r[...]
        u = prop_sum([pa_r, pb_r], [xsa_r, xsb_r]) * dinv
        h = jnp.maximum(
            jnp.dot(u, W1_r[...], preferred_element_type=jnp.float32)
            + b1_r[...], 0.0)
        v2 = jnp.dot(h, W2_r[...], preferred_element_type=jnp.float32) * dinv
        va_r[...] = v2[:, :96]
        vb_r[...] = v2[:, 96:]

    v2a, v2b = pl.pallas_call(
        tc1, grid=grid,
        in_specs=[row_spec(1), parts_spec(64), parts_spec(64),
                  row_spec(64), row_spec(64),
                  full_spec(W1), full_spec(b1r), full_spec(W2p)],
        out_specs=[row_spec(96), row_spec(80)],
        out_shape=[out_row(96), out_row(80)],
    )(dinv, p1a, p1b, xs1a, xs1b, W1, b1r, W2p)

    p2a = prop96(v2a, srcp, dstp)
    p2b = prop80(v2b, srcp, dstp)

    def tc2(dinv_r, pa_r, pb_r, va_r, vb_r, b_rr, W_rr, vo_r):
        dinv = dinv_r[...]
        s = prop_sum([pa_r, pb_r], [va_r, vb_r])
        h = jnp.maximum(s * dinv + b_rr[...], 0.0)
        vo_r[...] = jnp.dot(
            h, W_rr[...], preferred_element_type=jnp.float32) * dinv

    v3 = pl.pallas_call(
        tc2, grid=grid,
        in_specs=[row_spec(1), parts_spec(96), parts_spec(80),
                  row_spec(96), row_spec(80),
                  full_spec(b2p), full_spec(W3p)],
        out_specs=row_spec(96),
        out_shape=out_row(96),
    )(dinv, p2a, p2b, v2a, v2b, b2p, W3p)

    p3 = prop96(v3, srcp, dstp)

    def tc3(dinv_r, p_r, v_r, b_rr, W_rr, vo_r):
        dinv = dinv_r[...]
        h = jnp.maximum((p_r[0] + p_r[1] - v_r[...]) * dinv + b_rr[...], 0.0)
        vo_r[...] = jnp.dot(
            h, W_rr[...], preferred_element_type=jnp.float32) * dinv

    v4 = pl.pallas_call(
        tc3, grid=grid,
        in_specs=[row_spec(1), parts_spec(96), row_spec(96),
                  full_spec(b3p), full_spec(W4p)],
        out_specs=row_spec(48),
        out_shape=out_row(48),
    )(dinv, p3, v3, b3p, W4p)

    p4 = prop48(v4, srcp, dstp)

    def tc4(dinv_r, p_r, v_r, b_rr, v5_r):
        dinv = dinv_r[...]
        h = jnp.maximum((p_r[0] + p_r[1] - v_r[...]) * dinv + b_rr[...], 0.0)
        v5_r[...] = h * dinv

    v5 = pl.pallas_call(
        tc4, grid=grid,
        in_specs=[row_spec(1), parts_spec(48), row_spec(48),
                  full_spec(b4p)],
        out_specs=row_spec(48),
        out_shape=out_row(48),
    )(dinv, p4, v4, b4p)

    p5 = prop48(v5, srcp, dstp)

    def tc5(dinv_r, p_r, v_r, Wm_r, bm_r, Wl_r, bl_r, eps_r, pz_r, z_r):
        g = (p_r[0] + p_r[1] - v_r[...]) * dinv_r[...]
        mu = jnp.dot(g, Wm_r[...], preferred_element_type=jnp.float32) \
            + bm_r[...]
        ls = jnp.dot(g, Wl_r[...], preferred_element_type=jnp.float32) \
            + bl_r[...]
        z = mu + eps_r[...] * jnp.exp(ls)
        m = jnp.max(z, axis=1, keepdims=True)
        lse = m + jnp.log(jnp.sum(jnp.exp(z - m), axis=1, keepdims=True))
        pz_r[...] = z - lse
        z_r[...] = z

    pz, z = pl.pallas_call(
        tc5, grid=grid,
        in_specs=[row_spec(1), parts_spec(48), row_spec(48),
                  full_spec(W_mup), full_spec(b_mu[None, :]),
                  full_spec(W_lsp), full_spec(b_ls[None, :]),
                  row_spec(k_out)],
        out_specs=[row_spec(k_out), row_spec(k_out)],
        out_shape=[out_row(k_out), out_row(k_out)],
    )(dinv, p5, v5, W_mup, b_mu[None, :], W_lsp, b_ls[None, :], eps_p)

    return (pz[:n], z[:n])


# speedup vs baseline: 21.5614x; 1.0277x over previous
import functools

import jax
import jax.numpy as jnp
from jax import lax
from jax.experimental import pallas as pl
from jax.experimental.pallas import tpu as pltpu
from jax.experimental.pallas import tpu_sc as plsc

NC = 2
NS = 16
NW = NC * NS
CHUNK = 128

_MESH = plsc.VectorSubcoreMesh(core_axis_name="c", subcore_axis_name="s")
_SC_PARAMS = pltpu.CompilerParams(use_tc_tiling_on_sc=False)


def _make_prop(np_rows, d, ep_w, gk, chunk=CHUNK):
    rpt = np_rows // NS
    n_chunks = ep_w // chunk

    @functools.partial(
        pl.kernel,
        out_type=jax.ShapeDtypeStruct((2, np_rows, d), jnp.float32),
        mesh=_MESH,
        scratch_types=(
            [pltpu.VMEM((ep_w,), jnp.int32)] * 2
            + [pltpu.VMEM((chunk, d), jnp.float32)] * gk
            + [pltpu.VMEM((chunk,), jnp.int32)] * (2 * gk)
            + [pltpu.VMEM_SHARED((np_rows, d), jnp.float32)]
            + [pltpu.SemaphoreType.DMA] * 3
        ),
        compiler_params=_SC_PARAMS,
    )
    def prop(xin, srcp, dstp, out, *refs):
        src_all, dst_all = refs[0], refs[1]
        rows = refs[2:2 + gk]
        sv = refs[2 + gk:2 + 2 * gk]
        dv = refs[2 + 2 * gk:2 + 3 * gk]
        acc = refs[2 + 3 * gk]
        sem_i, sem_g, sem_s = refs[3 + 3 * gk:6 + 3 * gk]
        cid = lax.axis_index("c")
        sid = lax.axis_index("s")
        wid = cid * NS + sid
        r0 = sid * rpt

        ebase = wid * ep_w
        ic0 = pltpu.async_copy(srcp.at[pl.ds(ebase, ep_w)], src_all, sem_i)
        ic1 = pltpu.async_copy(dstp.at[pl.ds(ebase, ep_w)], dst_all, sem_i)
        n_init = rpt // chunk
        descs = {}
        for i in range(min(2, n_init)):
            descs[i] = pltpu.async_copy(
                xin.at[pl.ds(r0 + i * chunk, chunk)], rows[i % 2], sem_g)
        for i in range(n_init):
            descs[i].wait()
            pltpu.sync_copy(rows[i % 2], acc.at[pl.ds(r0 + i * chunk, chunk)])
            if i + 2 < n_init:
                descs[i + 2] = pltpu.async_copy(
                    xin.at[pl.ds(r0 + (i + 2) * chunk, chunk)],
                    rows[i % 2], sem_g)
        ic0.wait()
        ic1.wait()
        plsc.subcore_barrier()

        def body(j, carry):
            gd = []
            for k in range(gk):
                cb = (gk * j + k) * chunk

                @pl.when(j >= 1)
                def _():
                    pltpu.make_async_copy(
                        rows[k], acc.at[dv[k]], sem_s).wait()

                for t in range(chunk // 16):
                    sv[k][pl.ds(16 * t, 16)] = src_all[pl.ds(cb + 16 * t, 16)]
                    dv[k][pl.ds(16 * t, 16)] = dst_all[pl.ds(cb + 16 * t, 16)]
                gd.append(pltpu.async_copy(xin.at[sv[k]], rows[k], sem_g))
            for k in range(gk):
                gd[k].wait()
                pltpu.async_copy(rows[k], acc.at[dv[k]], sem_s, add=True)
            return carry

        lax.fori_loop(0, n_chunks // gk, body, 0)
        for k in range(gk):
            pltpu.make_async_copy(rows[k], acc.at[dv[k]], sem_s).wait()
        plsc.subcore_barrier()

        odescs = {}
        for i in range(n_init):
            if i >= 2:
                odescs[i - 2].wait()
            rb = r0 + i * chunk
            pltpu.sync_copy(acc.at[pl.ds(rb, chunk)], rows[i % 2])
            odescs[i] = pltpu.async_copy(
                rows[i % 2], out.at[cid, pl.ds(rb, chunk)], sem_g)
        for i in range(max(0, n_init - 2), n_init):
            odescs[i].wait()

    return prop


def _make_deg(np_rows, degw, ep_w, n_chunks):
    rpt = np_rows // NS

    @functools.partial(
        pl.kernel,
        out_type=jax.ShapeDtypeStruct((2, np_rows, degw), jnp.float32),
        mesh=_MESH,
        scratch_types=[
            pltpu.VMEM((ep_w // CHUNK, CHUNK), jnp.int32),
            pltpu.VMEM((CHUNK, degw), jnp.float32),
            pltpu.VMEM_SHARED((np_rows, degw), jnp.float32),
            pltpu.SemaphoreType.DMA,
            pltpu.SemaphoreType.DMA,
        ],
        compiler_params=_SC_PARAMS,
    )
    def deg(ones_hbm, dstp, out, dst_all, ones_v, acc, sem_i, sem_s):
        cid = lax.axis_index("c")
        sid = lax.axis_index("s")
        wid = cid * NS + sid
        r0 = sid * rpt
        ic = pltpu.async_copy(dstp.at[wid], dst_all, sem_i)
        pltpu.sync_copy(ones_hbm, ones_v)

        def init_body(i, carry):
            pltpu.sync_copy(ones_v, acc.at[pl.ds(r0 + i * CHUNK, CHUNK)])
            return carry

        lax.fori_loop(0, rpt // CHUNK, init_body, 0)
        ic.wait()
        plsc.subcore_barrier()

        def edge_body(i, carry):
            sd = []
            for k in range(8):
                sd.append(pltpu.async_copy(
                    ones_v, acc.at[dst_all.at[8 * i + k]], sem_s, add=True))
            for k in range(8):
                sd[k].wait()
            return carry

        lax.fori_loop(0, n_chunks // 8, edge_body, 0)
        plsc.subcore_barrier()

        def out_body(i, carry):
            rb = r0 + i * CHUNK
            pltpu.sync_copy(acc.at[pl.ds(rb, CHUNK)], ones_v)
            pltpu.sync_copy(ones_v, out.at[cid, pl.ds(rb, CHUNK)])
            return carry

        lax.fori_loop(0, rpt // CHUNK, out_body, 0)

    return deg


def _pad2(a, rows, cols):
    return jnp.pad(a, ((0, rows - a.shape[0]), (0, cols - a.shape[1])))


def kernel(x, edge_index, eps, W1, b1, W2, b2, W3, b3, W4, b4,
           W_mu, b_mu, W_ls, b_ls):
    n = x.shape[0]
    e = edge_index.shape[1]
    f_in = x.shape[1]
    k_out = W_mu.shape[1]
    np_rows = (n + 16 * CHUNK - 1) // (16 * CHUNK) * (16 * CHUNK)
    br = 1024
    grid = (np_rows // br,)

    gchunk = 40 * CHUNK
    ep_w = ((e + NW - 1) // NW + gchunk - 1) // gchunk * gchunk
    e_pad = ep_w * NW
    pad = e_pad - e
    pad_i = jnp.arange(pad, dtype=jnp.int32)
    srcp = jnp.concatenate([edge_index[0], n + (pad_i % 8)])
    dstp = jnp.concatenate([edge_index[1], n + 8 + (pad_i % 32)])
    n_chunks = ep_w // CHUNK
    dstp3 = dstp.reshape(NW, n_chunks, CHUNK)
    xpad = _pad2(x, np_rows, f_in)
    eps_p = _pad2(eps, np_rows, k_out)
    W2p = _pad2(W2, W2.shape[0], 176)
    W3p = _pad2(W3, 176, 96)
    W4p = _pad2(W4, 96, 48)
    W_mup = _pad2(W_mu, 48, k_out)
    W_lsp = _pad2(W_ls, 48, k_out)
    b1r = b1[None, :]
    b2p = _pad2(b2[None, :], 1, 176)
    b3p = _pad2(b3[None, :], 1, 96)
    b4p = _pad2(b4[None, :], 1, 48)
    ones = jnp.ones((CHUNK, 8), dtype=jnp.float32)

    row_spec = lambda w: pl.BlockSpec((br, w), lambda i: (i, 0))
    parts_spec = lambda w: pl.BlockSpec((2, br, w), lambda i: (0, i, 0))
    full_spec = lambda a: pl.BlockSpec(a.shape, lambda i: (0,) * a.ndim)
    out_row = lambda w: jax.ShapeDtypeStruct((np_rows, w), jnp.float32)

    prop64 = _make_prop(np_rows, 64, ep_w, 5)
    prop48 = _make_prop(np_rows, 48, ep_w, 10)
    prop80 = _make_prop(np_rows, 80, ep_w, 4)
    prop96 = _make_prop(np_rows, 96, ep_w, 5, chunk=64)

    def prop_sum(parts, vs):
        return jnp.concatenate(
            [p[0] + p[1] - v[...] for p, v in zip(parts, vs)], axis=1)

    degp = _make_deg(np_rows, 8, ep_w, n_chunks)(ones, dstp3)

    def tc0(degp_r, x_r, dinv_r, xsa_r, xsb_r):
        i = pl.program_id(0)
        deg = degp_r[0, :, 0:1] + degp_r[1, :, 0:1] - 1.0
        rows = i * br + lax.broadcasted_iota(jnp.int32, (br, 1), 0)
        dinv = jnp.where(rows < n, lax.rsqrt(deg), 0.0)
        dinv_r[...] = dinv
        xs = x_r[...] * dinv
        xsa_r[...] = xs[:, :64]
        xsb_r[...] = xs[:, 64:]

    dinv, xs1a, xs1b = pl.pallas_call(
        tc0, grid=grid,
        in_specs=[parts_spec(8), row_spec(f_in)],
        out_specs=[row_spec(1), row_spec(64), row_spec(64)],
        out_shape=[out_row(1), out_row(64), out_row(64)],
    )(degp, xpad)

    p1a = prop64(xs1a, srcp, dstp)
    p1b = prop64(xs1b, srcp, dstp)

    def tc1(dinv_r, pa_r, pb_r, xsa_r, xsb_r, W1_r, b1_r, W2_r,
            va_r, vb_r):
        dinv = dinv_r[...]
        u = prop_sum([pa_r, pb_r], [xsa_r, xsb_r]) * dinv
        h = jnp.maximum(
            jnp.dot(u, W1_r[...], preferred_element_type=jnp.float32)
            + b1_r[...], 0.0)
        v2 = jnp.dot(h, W2_r[...], preferred_element_type=jnp.float32) * dinv
        va_r[...] = v2[:, :96]
        vb_r[...] = v2[:, 96:]

    v2a, v2b = pl.pallas_call(
        tc1, grid=grid,
        in_specs=[row_spec(1), parts_spec(64), parts_spec(64),
                  row_spec(64), row_spec(64),
                  full_spec(W1), full_spec(b1r), full_spec(W2p)],
        out_specs=[row_spec(96), row_spec(80)],
        out_shape=[out_row(96), out_row(80)],
    )(dinv, p1a, p1b, xs1a, xs1b, W1, b1r, W2p)

    p2a = prop96(v2a, srcp, dstp)
    p2b = prop80(v2b, srcp, dstp)

    def tc2(dinv_r, pa_r, pb_r, va_r, vb_r, b_rr, W_rr, vo_r):
        dinv = dinv_r[...]
        s = prop_sum([pa_r, pb_r], [va_r, vb_r])
        h = jnp.maximum(s * dinv + b_rr[...], 0.0)
        vo_r[...] = jnp.dot(
            h, W_rr[...], preferred_element_type=jnp.float32) * dinv

    v3 = pl.pallas_call(
        tc2, grid=grid,
        in_specs=[row_spec(1), parts_spec(96), parts_spec(80),
                  row_spec(96), row_spec(80),
                  full_spec(b2p), full_spec(W3p)],
        out_specs=row_spec(96),
        out_shape=out_row(96),
    )(dinv, p2a, p2b, v2a, v2b, b2p, W3p)

    p3 = prop96(v3, srcp, dstp)

    def tc3(dinv_r, p_r, v_r, b_rr, W_rr, vo_r):
        dinv = dinv_r[...]
        h = jnp.maximum((p_r[0] + p_r[1] - v_r[...]) * dinv + b_rr[...], 0.0)
        vo_r[...] = jnp.dot(
            h, W_rr[...], preferred_element_type=jnp.float32) * dinv

    v4 = pl.pallas_call(
        tc3, grid=grid,
        in_specs=[row_spec(1), parts_spec(96), row_spec(96),
                  full_spec(b3p), full_spec(W4p)],
        out_specs=row_spec(48),
        out_shape=out_row(48),
    )(dinv, p3, v3, b3p, W4p)

    p4 = prop48(v4, srcp, dstp)

    def tc4(dinv_r, p_r, v_r, b_rr, v5_r):
        dinv = dinv_r[...]
        h = jnp.maximum((p_r[0] + p_r[1] - v_r[...]) * dinv + b_rr[...], 0.0)
        v5_r[...] = h * dinv

    v5 = pl.pallas_call(
        tc4, grid=grid,
        in_specs=[row_spec(1), parts_spec(48), row_spec(48),
                  full_spec(b4p)],
        out_specs=row_spec(48),
        out_shape=out_row(48),
    )(dinv, p4, v4, b4p)

    p5 = prop48(v5, srcp, dstp)

    def tc5(dinv_r, p_r, v_r, Wm_r, bm_r, Wl_r, bl_r, eps_r, pz_r, z_r):
        g = (p_r[0] + p_r[1] - v_r[...]) * dinv_r[...]
        mu = jnp.dot(g, Wm_r[...], preferred_element_type=jnp.float32) \
            + bm_r[...]
        ls = jnp.dot(g, Wl_r[...], preferred_element_type=jnp.float32) \
            + bl_r[...]
        z = mu + eps_r[...] * jnp.exp(ls)
        m = jnp.max(z, axis=1, keepdims=True)
        lse = m + jnp.log(jnp.sum(jnp.exp(z - m), axis=1, keepdims=True))
        pz_r[...] = z - lse
        z_r[...] = z

    pz, z = pl.pallas_call(
        tc5, grid=grid,
        in_specs=[row_spec(1), parts_spec(48), row_spec(48),
                  full_spec(W_mup), full_spec(b_mu[None, :]),
                  full_spec(W_lsp), full_spec(b_ls[None, :]),
                  row_spec(k_out)],
        out_specs=[row_spec(k_out), row_spec(k_out)],
        out_shape=[out_row(k_out), out_row(k_out)],
    )(dinv, p5, v5, W_mup, b_mu[None, :], W_lsp, b_ls[None, :], eps_p)

    return (pz[:n], z[:n])
